# Initial kernel scaffold; baseline (speedup 1.0000x reference)
import jax
import jax.numpy as jnp
from jax.experimental import pallas as pl


def kernel(edge_index, y_hat):
    raise NotImplementedError("write your pallas kernel here")


# speedup vs baseline: 28.2343x; 28.2343x over previous
import dataclasses
import functools

import jax
import jax.numpy as jnp
from jax import lax
from jax.experimental import pallas as pl
from jax.experimental.pallas import tpu as pltpu
from jax.experimental.pallas import tpu_sc as plsc

N_PAD = 100352
NC, NS, L = 2, 16, 16
NW = NC * NS
E_TOTAL = 6400000
EPW = E_TOTAL // NW
CE = 2000
NCH = EPW // CE
SLICE = N_PAD // NS


def _sc_compiler_params():
    cp = pltpu.CompilerParams()
    if "needs_layout_passes" in pltpu.CompilerParams.__dataclass_fields__:
        cp = dataclasses.replace(cp, needs_layout_passes=False)
    return cp


def _sc_scatter(edge_index, y):
    mesh = plsc.VectorSubcoreMesh(core_axis_name="c", subcore_axis_name="s")

    @functools.partial(
        pl.kernel,
        compiler_params=_sc_compiler_params(),
        out_type=(
            jax.ShapeDtypeStruct((NC, N_PAD), jnp.float32),
            jax.ShapeDtypeStruct((NW, L), jnp.int32),
        ),
        mesh=mesh,
        scratch_types=[
            pltpu.VMEM((N_PAD,), jnp.float32),
            pltpu.VMEM((CE,), jnp.int32),
            pltpu.VMEM((CE,), jnp.int32),
            pltpu.VMEM((CE,), jnp.float32),
            pltpu.VMEM((L,), jnp.int32),
            pltpu.VMEM((SLICE,), jnp.float32),
            pltpu.VMEM((SLICE,), jnp.float32),
            pltpu.VMEM_SHARED((NS, SLICE), jnp.float32),
        ],
    )
    def k(ei_hbm, y_hbm, part_hbm, max_hbm,
          acc, sbuf, dbuf, ybuf, maxb, tmp, ssum, shared):
        cid = lax.axis_index("c")
        sid = lax.axis_index("s")
        wid = cid * NS + sid

        zero16 = jnp.zeros((L,), jnp.float32)

        @pl.loop(0, N_PAD, step=L)
        def _(i):
            acc[pl.ds(i, L)] = zero16

        maxb[...] = jnp.zeros((L,), jnp.int32)

        ebase = wid * EPW

        @pl.loop(0, NCH)
        def _(c):
            base = ebase + c * CE
            pltpu.sync_copy(ei_hbm.at[pl.ds(base, CE)], sbuf)
            pltpu.sync_copy(ei_hbm.at[pl.ds(E_TOTAL + base, CE)], dbuf)
            pltpu.sync_copy(y_hbm.at[pl.ds(base, CE)], ybuf)

            @pl.loop(0, CE, step=L)
            def _(j):
                s = sbuf[pl.ds(j, L)]
                d = dbuf[pl.ds(j, L)]
                yv = ybuf[pl.ds(j, L)]
                plsc.addupdate_scatter(acc, [d], yv)
                plsc.addupdate_scatter(acc, [s], -yv)
                maxb[...] = jnp.maximum(maxb[...], jnp.maximum(s, d))

        lo = sid * SLICE

        @pl.loop(0, SLICE, step=L)
        def _(i):
            ssum[pl.ds(i, L)] = acc[pl.ds(lo + i, L)]

        @pl.loop(1, NS)
        def _(r):
            dst_owner = lax.rem(sid + r, NS)
            pltpu.sync_copy(acc.at[pl.ds(dst_owner * SLICE, SLICE)],
                            shared.at[sid])
            plsc.subcore_barrier()
            src_slot = lax.rem(sid - r + NS, NS)
            pltpu.sync_copy(shared.at[src_slot], tmp)

            @pl.loop(0, SLICE, step=L)
            def _(i):
                ssum[pl.ds(i, L)] += tmp[pl.ds(i, L)]

            plsc.subcore_barrier()

        pltpu.sync_copy(ssum, part_hbm.at[cid, pl.ds(lo, SLICE)])
        pltpu.sync_copy(maxb, max_hbm.at[wid])

    return k(edge_index, y)


def _tc_finalize(partials, maxes):
    def body(p_ref, m_ref, o_ref):
        a = p_ref[...]
        diff = a[: N_PAD // 128] + a[N_PAD // 128:]
        m = jnp.max(m_ref[...])
        o_ref[0, 0] = jnp.sum(jnp.abs(diff)) / (m.astype(jnp.float32) + 1.0)

    p2 = partials.reshape(NC * (N_PAD // 128), 128)
    m2 = maxes.reshape(NW * L // 128, 128)
    return pl.pallas_call(
        body,
        out_shape=jax.ShapeDtypeStruct((1, 1), jnp.float32),
        out_specs=pl.BlockSpec(memory_space=pltpu.SMEM),
    )(p2, m2)


def kernel(edge_index, y_hat):
    y = y_hat.reshape(-1)
    partials, maxes = _sc_scatter(edge_index.reshape(-1), y)
    return _tc_finalize(partials, maxes)[0, 0]


# speedup vs baseline: 54.6572x; 1.9358x over previous
import dataclasses
import functools

import jax
import jax.numpy as jnp
from jax import lax
from jax.experimental import pallas as pl
from jax.experimental.pallas import tpu as pltpu
from jax.experimental.pallas import tpu_sc as plsc

N_PAD = 100352
NC, NS, L = 2, 16, 16
NW = NC * NS
E_TOTAL = 6400000
EPW = E_TOTAL // NW
CE = 1000
NCH = EPW // CE
SLICE = N_PAD // NS


def _sc_compiler_params():
    cp = pltpu.CompilerParams()
    if "needs_layout_passes" in pltpu.CompilerParams.__dataclass_fields__:
        cp = dataclasses.replace(cp, needs_layout_passes=False)
    return cp


def _sc_scatter(edge_index, y):
    mesh = plsc.VectorSubcoreMesh(core_axis_name="c", subcore_axis_name="s")

    @functools.partial(
        pl.kernel,
        compiler_params=_sc_compiler_params(),
        out_type=(
            jax.ShapeDtypeStruct((NC, N_PAD), jnp.float32),
            jax.ShapeDtypeStruct((NW, L), jnp.int32),
        ),
        mesh=mesh,
        scratch_types=[
            pltpu.VMEM((N_PAD,), jnp.float32),
            pltpu.VMEM((CE,), jnp.int32),
            pltpu.VMEM((CE,), jnp.int32),
            pltpu.VMEM((CE,), jnp.float32),
            pltpu.VMEM((CE,), jnp.int32),
            pltpu.VMEM((CE,), jnp.int32),
            pltpu.VMEM((CE,), jnp.float32),
            pltpu.VMEM((L,), jnp.int32),
            pltpu.VMEM((SLICE,), jnp.float32),
            pltpu.VMEM((SLICE,), jnp.float32),
            pltpu.VMEM_SHARED((NS, SLICE), jnp.float32),
            pltpu.SemaphoreType.DMA,
            pltpu.SemaphoreType.DMA,
        ],
    )
    def k(ei_hbm, y_hbm, part_hbm, max_hbm,
          acc, sbuf0, dbuf0, ybuf0, sbuf1, dbuf1, ybuf1,
          maxb, tmp, ssum, shared, sem0, sem1):
        cid = lax.axis_index("c")
        sid = lax.axis_index("s")
        wid = cid * NS + sid

        zero16 = jnp.zeros((L,), jnp.float32)

        @pl.loop(0, N_PAD, step=L)
        def _(i):
            acc[pl.ds(i, L)] = zero16

        maxb[...] = jnp.zeros((L,), jnp.int32)

        ebase = wid * EPW

        def start(c, sb, db, yb, sem):
            base = ebase + c * CE
            pltpu.async_copy(ei_hbm.at[pl.ds(base, CE)], sb, sem)
            pltpu.async_copy(ei_hbm.at[pl.ds(E_TOTAL + base, CE)], db, sem)
            pltpu.async_copy(y_hbm.at[pl.ds(base, CE)], yb, sem)

        def wait(sb, db, yb, sem):
            pltpu.make_async_copy(ei_hbm.at[pl.ds(0, CE)], sb, sem).wait()
            pltpu.make_async_copy(ei_hbm.at[pl.ds(0, CE)], db, sem).wait()
            pltpu.make_async_copy(y_hbm.at[pl.ds(0, CE)], yb, sem).wait()

        def process(sb, db, yb):
            def group(g, mv):
                j = g * L
                s = sb[pl.ds(j, L)]
                d = db[pl.ds(j, L)]
                yv = yb[pl.ds(j, L)]
                plsc.addupdate_scatter(acc, [d], yv)
                plsc.addupdate_scatter(acc, [s], -yv)
                return jnp.maximum(mv, jnp.maximum(s, d))

            maxb[...] = lax.fori_loop(0, CE // L, group, maxb[...])

        start(0, sbuf0, dbuf0, ybuf0, sem0)
        start(1, sbuf1, dbuf1, ybuf1, sem1)

        @pl.loop(0, NCH, step=2)
        def _(c):
            wait(sbuf0, dbuf0, ybuf0, sem0)
            process(sbuf0, dbuf0, ybuf0)

            @pl.when(c + 2 < NCH)
            def _():
                start(c + 2, sbuf0, dbuf0, ybuf0, sem0)

            wait(sbuf1, dbuf1, ybuf1, sem1)
            process(sbuf1, dbuf1, ybuf1)

            @pl.when(c + 3 < NCH)
            def _():
                start(c + 3, sbuf1, dbuf1, ybuf1, sem1)

        lo = sid * SLICE

        @pl.loop(0, SLICE, step=L)
        def _(i):
            ssum[pl.ds(i, L)] = acc[pl.ds(lo + i, L)]

        @pl.loop(1, NS)
        def _(r):
            dst_owner = lax.rem(sid + r, NS)
            pltpu.sync_copy(acc.at[pl.ds(dst_owner * SLICE, SLICE)],
                            shared.at[sid])
            plsc.subcore_barrier()
            src_slot = lax.rem(sid - r + NS, NS)
            pltpu.sync_copy(shared.at[src_slot], tmp)

            @pl.loop(0, SLICE, step=L)
            def _(i):
                ssum[pl.ds(i, L)] += tmp[pl.ds(i, L)]

            plsc.subcore_barrier()

        pltpu.sync_copy(ssum, part_hbm.at[cid, pl.ds(lo, SLICE)])
        pltpu.sync_copy(maxb, max_hbm.at[wid])

    return k(edge_index, y)


def _tc_finalize(partials, maxes):
    def body(p_ref, m_ref, o_ref):
        a = p_ref[...]
        diff = a[: N_PAD // 128] + a[N_PAD // 128:]
        m = jnp.max(m_ref[...])
        o_ref[0, 0] = jnp.sum(jnp.abs(diff)) / (m.astype(jnp.float32) + 1.0)

    p2 = partials.reshape(NC * (N_PAD // 128), 128)
    m2 = maxes.reshape(NW * L // 128, 128)
    return pl.pallas_call(
        body,
        out_shape=jax.ShapeDtypeStruct((1, 1), jnp.float32),
        out_specs=pl.BlockSpec(memory_space=pltpu.SMEM),
    )(p2, m2)


def kernel(edge_index, y_hat):
    y = y_hat.reshape(-1)
    partials, maxes = _sc_scatter(edge_index.reshape(-1), y)
    return _tc_finalize(partials, maxes)[0, 0]


# speedup vs baseline: 66.8263x; 1.2226x over previous
import dataclasses
import functools

import jax
import jax.numpy as jnp
from jax import lax
from jax.experimental import pallas as pl
from jax.experimental.pallas import tpu as pltpu
from jax.experimental.pallas import tpu_sc as plsc

N_PAD = 100352
NC, NS, L = 2, 16, 16
NW = NC * NS
E_TOTAL = 6400000
EPW = E_TOTAL // NW
CE = 1000
NCH = EPW // CE
SLICE = N_PAD // NS


def _sc_compiler_params():
    cp = pltpu.CompilerParams()
    if "needs_layout_passes" in pltpu.CompilerParams.__dataclass_fields__:
        cp = dataclasses.replace(cp, needs_layout_passes=False)
    return cp


def _sc_scatter(edge_index, y):
    mesh = plsc.VectorSubcoreMesh(core_axis_name="c", subcore_axis_name="s")

    @functools.partial(
        pl.kernel,
        compiler_params=_sc_compiler_params(),
        out_type=(
            jax.ShapeDtypeStruct((NC, N_PAD), jnp.float32),
            jax.ShapeDtypeStruct((NW, L), jnp.int32),
        ),
        mesh=mesh,
        scratch_types=[
            pltpu.VMEM((N_PAD,), jnp.float32),
            pltpu.VMEM((CE,), jnp.int32),
            pltpu.VMEM((CE,), jnp.int32),
            pltpu.VMEM((CE,), jnp.float32),
            pltpu.VMEM((CE,), jnp.int32),
            pltpu.VMEM((CE,), jnp.int32),
            pltpu.VMEM((CE,), jnp.float32),
            pltpu.VMEM((L,), jnp.int32),
            pltpu.VMEM((SLICE,), jnp.float32),
            pltpu.VMEM((SLICE,), jnp.float32),
            pltpu.VMEM_SHARED((NS, SLICE), jnp.float32),
            pltpu.SemaphoreType.DMA,
            pltpu.SemaphoreType.DMA,
        ],
    )
    def k(ei_hbm, y_hbm, part_hbm, max_hbm,
          acc, sbuf0, dbuf0, ybuf0, sbuf1, dbuf1, ybuf1,
          maxb, tmp, ssum, shared, sem0, sem1):
        cid = lax.axis_index("c")
        sid = lax.axis_index("s")
        wid = cid * NS + sid

        zero16 = jnp.zeros((L,), jnp.float32)

        @pl.loop(0, N_PAD, step=8 * L)
        def _(i):
            for u in range(8):
                acc[pl.ds(i + u * L, L)] = zero16

        maxb[...] = jnp.zeros((L,), jnp.int32)

        ebase = wid * EPW

        def start(c, sb, db, yb, sem):
            base = ebase + c * CE
            pltpu.async_copy(ei_hbm.at[pl.ds(base, CE)], sb, sem)
            pltpu.async_copy(ei_hbm.at[pl.ds(E_TOTAL + base, CE)], db, sem)
            pltpu.async_copy(y_hbm.at[pl.ds(base, CE)], yb, sem)

        def wait(sb, db, yb, sem):
            pltpu.make_async_copy(ei_hbm.at[pl.ds(0, CE)], sb, sem).wait()
            pltpu.make_async_copy(ei_hbm.at[pl.ds(0, CE)], db, sem).wait()
            pltpu.make_async_copy(y_hbm.at[pl.ds(0, CE)], yb, sem).wait()

        def process(sb, db, yb):
            def quad(q, mv):
                j = q * (4 * L)
                m = mv
                for u in range(4):
                    s = sb[pl.ds(j + u * L, L)]
                    d = db[pl.ds(j + u * L, L)]
                    yv = yb[pl.ds(j + u * L, L)]
                    plsc.addupdate_scatter(acc, [d], yv)
                    plsc.addupdate_scatter(acc, [s], -yv)
                    m = jnp.maximum(m, jnp.maximum(s, d))
                return m

            maxb[...] = lax.fori_loop(0, CE // (4 * L), quad, maxb[...])

        start(0, sbuf0, dbuf0, ybuf0, sem0)
        start(1, sbuf1, dbuf1, ybuf1, sem1)

        @pl.loop(0, NCH, step=2)
        def _(c):
            wait(sbuf0, dbuf0, ybuf0, sem0)
            process(sbuf0, dbuf0, ybuf0)

            @pl.when(c + 2 < NCH)
            def _():
                start(c + 2, sbuf0, dbuf0, ybuf0, sem0)

            wait(sbuf1, dbuf1, ybuf1, sem1)
            process(sbuf1, dbuf1, ybuf1)

            @pl.when(c + 3 < NCH)
            def _():
                start(c + 3, sbuf1, dbuf1, ybuf1, sem1)

        lo = sid * SLICE

        @pl.loop(0, SLICE, step=L)
        def _(i):
            ssum[pl.ds(i, L)] = acc[pl.ds(lo + i, L)]

        @pl.loop(1, NS)
        def _(r):
            dst_owner = lax.rem(sid + r, NS)
            pltpu.sync_copy(acc.at[pl.ds(dst_owner * SLICE, SLICE)],
                            shared.at[sid])
            plsc.subcore_barrier()
            src_slot = lax.rem(sid - r + NS, NS)
            pltpu.sync_copy(shared.at[src_slot], tmp)

            @pl.loop(0, SLICE, step=4 * L)
            def _(i):
                for u in range(4):
                    ssum[pl.ds(i + u * L, L)] += tmp[pl.ds(i + u * L, L)]

            plsc.subcore_barrier()

        pltpu.sync_copy(ssum, part_hbm.at[cid, pl.ds(lo, SLICE)])
        pltpu.sync_copy(maxb, max_hbm.at[wid])

    return k(edge_index, y)


def _tc_finalize(partials, maxes):
    def body(p_ref, m_ref, o_ref):
        a = p_ref[...]
        diff = a[: N_PAD // 128] + a[N_PAD // 128:]
        m = jnp.max(m_ref[...])
        o_ref[0, 0] = jnp.sum(jnp.abs(diff)) / (m.astype(jnp.float32) + 1.0)

    p2 = partials.reshape(NC * (N_PAD // 128), 128)
    m2 = maxes.reshape(NW * L // 128, 128)
    return pl.pallas_call(
        body,
        out_shape=jax.ShapeDtypeStruct((1, 1), jnp.float32),
        out_specs=pl.BlockSpec(memory_space=pltpu.SMEM),
    )(p2, m2)


def kernel(edge_index, y_hat):
    y = y_hat.reshape(-1)
    partials, maxes = _sc_scatter(edge_index.reshape(-1), y)
    return _tc_finalize(partials, maxes)[0, 0]


# speedup vs baseline: 73.5169x; 1.1001x over previous
import dataclasses
import functools

import jax
import jax.numpy as jnp
from jax import lax
from jax.experimental import pallas as pl
from jax.experimental.pallas import tpu as pltpu
from jax.experimental.pallas import tpu_sc as plsc

N_PAD = 100352
NC, NS, L = 2, 16, 16
NW = NC * NS
E_TOTAL = 6400000
EPW = E_TOTAL // NW
CE = 1600
NCH = EPW // CE
SLICE = N_PAD // NS


def _sc_compiler_params():
    cp = pltpu.CompilerParams()
    if "needs_layout_passes" in pltpu.CompilerParams.__dataclass_fields__:
        cp = dataclasses.replace(cp, needs_layout_passes=False)
    return cp


def _sc_scatter(edge_index, y):
    mesh = plsc.VectorSubcoreMesh(core_axis_name="c", subcore_axis_name="s")

    @functools.partial(
        pl.kernel,
        compiler_params=_sc_compiler_params(),
        out_type=(
            jax.ShapeDtypeStruct((NC, N_PAD), jnp.float32),
            jax.ShapeDtypeStruct((NW, L), jnp.int32),
        ),
        mesh=mesh,
        scratch_types=[
            pltpu.VMEM((N_PAD,), jnp.float32),
            pltpu.VMEM((CE,), jnp.int32),
            pltpu.VMEM((CE,), jnp.int32),
            pltpu.VMEM((CE,), jnp.float32),
            pltpu.VMEM((CE,), jnp.int32),
            pltpu.VMEM((CE,), jnp.int32),
            pltpu.VMEM((CE,), jnp.float32),
            pltpu.VMEM((L,), jnp.int32),
            pltpu.VMEM((SLICE,), jnp.float32),
            pltpu.VMEM((SLICE,), jnp.float32),
            pltpu.VMEM_SHARED((NS, SLICE), jnp.float32),
            pltpu.SemaphoreType.DMA,
            pltpu.SemaphoreType.DMA,
        ],
    )
    def k(ei_hbm, y_hbm, part_hbm, max_hbm,
          acc, sbuf0, dbuf0, ybuf0, sbuf1, dbuf1, ybuf1,
          maxb, tmp, ssum, shared, sem0, sem1):
        cid = lax.axis_index("c")
        sid = lax.axis_index("s")
        wid = cid * NS + sid

        zero16 = jnp.zeros((L,), jnp.float32)

        @pl.loop(0, N_PAD, step=8 * L)
        def _(i):
            for u in range(8):
                acc[pl.ds(i + u * L, L)] = zero16

        maxb[...] = jnp.zeros((L,), jnp.int32)

        ebase = wid * EPW

        def start(c, sb, db, yb, sem):
            base = ebase + c * CE
            pltpu.async_copy(ei_hbm.at[pl.ds(base, CE)], sb, sem)
            pltpu.async_copy(ei_hbm.at[pl.ds(E_TOTAL + base, CE)], db, sem)
            pltpu.async_copy(y_hbm.at[pl.ds(base, CE)], yb, sem)

        def wait(sb, db, yb, sem):
            pltpu.make_async_copy(ei_hbm.at[pl.ds(0, CE)], sb, sem).wait()
            pltpu.make_async_copy(ei_hbm.at[pl.ds(0, CE)], db, sem).wait()
            pltpu.make_async_copy(y_hbm.at[pl.ds(0, CE)], yb, sem).wait()

        def process(sb, db, yb):
            def quad(q, mv):
                j = q * (4 * L)
                m = mv
                for u in range(4):
                    s = sb[pl.ds(j + u * L, L)]
                    d = db[pl.ds(j + u * L, L)]
                    yv = yb[pl.ds(j + u * L, L)]
                    plsc.addupdate_scatter(acc, [d], yv)
                    plsc.addupdate_scatter(acc, [s], -yv)
                    m = jnp.maximum(m, jnp.maximum(s, d))
                return m

            assert CE % (4 * L) == 0
            maxb[...] = lax.fori_loop(0, CE // (4 * L), quad, maxb[...])

        start(0, sbuf0, dbuf0, ybuf0, sem0)
        start(1, sbuf1, dbuf1, ybuf1, sem1)

        @pl.loop(0, NCH, step=2)
        def _(c):
            wait(sbuf0, dbuf0, ybuf0, sem0)
            process(sbuf0, dbuf0, ybuf0)

            @pl.when(c + 2 < NCH)
            def _():
                start(c + 2, sbuf0, dbuf0, ybuf0, sem0)

            @pl.when(c + 1 < NCH)
            def _():
                wait(sbuf1, dbuf1, ybuf1, sem1)
                process(sbuf1, dbuf1, ybuf1)

                @pl.when(c + 3 < NCH)
                def _():
                    start(c + 3, sbuf1, dbuf1, ybuf1, sem1)

        lo = sid * SLICE

        @pl.loop(0, SLICE, step=L)
        def _(i):
            ssum[pl.ds(i, L)] = acc[pl.ds(lo + i, L)]

        @pl.loop(1, NS)
        def _(r):
            dst_owner = lax.rem(sid + r, NS)
            pltpu.sync_copy(acc.at[pl.ds(dst_owner * SLICE, SLICE)],
                            shared.at[sid])
            plsc.subcore_barrier()
            src_slot = lax.rem(sid - r + NS, NS)
            pltpu.sync_copy(shared.at[src_slot], tmp)

            @pl.loop(0, SLICE, step=4 * L)
            def _(i):
                for u in range(4):
                    ssum[pl.ds(i + u * L, L)] += tmp[pl.ds(i + u * L, L)]

            plsc.subcore_barrier()

        pltpu.sync_copy(ssum, part_hbm.at[cid, pl.ds(lo, SLICE)])
        pltpu.sync_copy(maxb, max_hbm.at[wid])

    return k(edge_index, y)


def _tc_finalize(partials, maxes):
    def body(p_ref, m_ref, o_ref):
        a = p_ref[...]
        diff = a[: N_PAD // 128] + a[N_PAD // 128:]
        m = jnp.max(m_ref[...])
        o_ref[0, 0] = jnp.sum(jnp.abs(diff)) / (m.astype(jnp.float32) + 1.0)

    p2 = partials.reshape(NC * (N_PAD // 128), 128)
    m2 = maxes.reshape(NW * L // 128, 128)
    return pl.pallas_call(
        body,
        out_shape=jax.ShapeDtypeStruct((1, 1), jnp.float32),
        out_specs=pl.BlockSpec(memory_space=pltpu.SMEM),
    )(p2, m2)


def kernel(edge_index, y_hat):
    y = y_hat.reshape(-1)
    partials, maxes = _sc_scatter(edge_index.reshape(-1), y)
    return _tc_finalize(partials, maxes)[0, 0]


# speedup vs baseline: 89.7324x; 1.2206x over previous
import dataclasses
import functools

import jax
import jax.numpy as jnp
from jax import lax
from jax.experimental import pallas as pl
from jax.experimental.pallas import tpu as pltpu
from jax.experimental.pallas import tpu_sc as plsc

N_PAD = 100352
NC, NS, L = 2, 16, 16
NW = NC * NS
E_TOTAL = 6400000
EPW = 199936
CE = 1408
NCH = EPW // CE
TAIL_BASE = NW * EPW
TAIL_PER_TILE = (E_TOTAL - TAIL_BASE) // L
SLICE = N_PAD // NS


def _sc_compiler_params():
    cp = pltpu.CompilerParams()
    if "needs_layout_passes" in pltpu.CompilerParams.__dataclass_fields__:
        cp = dataclasses.replace(cp, needs_layout_passes=False)
    return cp


def _sc_scatter(edge_index, y):
    mesh = plsc.VectorSubcoreMesh(core_axis_name="c", subcore_axis_name="s")

    @functools.partial(
        pl.kernel,
        compiler_params=_sc_compiler_params(),
        out_type=(
            jax.ShapeDtypeStruct((NC, N_PAD), jnp.float32),
            jax.ShapeDtypeStruct((NW, L), jnp.int32),
        ),
        mesh=mesh,
        scratch_types=[
            pltpu.VMEM((N_PAD,), jnp.float32),
            pltpu.VMEM((2, CE), jnp.int32),
            pltpu.VMEM((CE,), jnp.float32),
            pltpu.VMEM((2, CE), jnp.int32),
            pltpu.VMEM((CE,), jnp.float32),
            pltpu.VMEM((L,), jnp.int32),
            pltpu.VMEM((SLICE,), jnp.float32),
            pltpu.VMEM((SLICE,), jnp.float32),
            pltpu.VMEM_SHARED((NS, SLICE), jnp.float32),
            pltpu.SemaphoreType.DMA,
            pltpu.SemaphoreType.DMA,
        ],
    )
    def k(ei_hbm, y_hbm, part_hbm, max_hbm,
          acc, ebuf0, ybuf0, ebuf1, ybuf1,
          maxb, tmp, ssum, shared, sem0, sem1):
        cid = lax.axis_index("c")
        sid = lax.axis_index("s")
        wid = cid * NS + sid

        zero16 = jnp.zeros((L,), jnp.float32)

        @pl.loop(0, N_PAD, step=8 * L)
        def _(i):
            for u in range(8):
                acc[pl.ds(i + u * L, L)] = zero16

        maxb[...] = jnp.zeros((L,), jnp.int32)

        ebase = wid * EPW

        def start(c, eb, yb, sem):
            base = pl.multiple_of(ebase + c * CE, 128)
            pltpu.async_copy(ei_hbm.at[:, pl.ds(base, CE)], eb, sem)
            pltpu.async_copy(y_hbm.at[pl.ds(base, CE)], yb, sem)

        def wait(eb, yb, sem):
            pltpu.make_async_copy(ei_hbm.at[:, pl.ds(0, CE)], eb, sem).wait()
            pltpu.make_async_copy(y_hbm.at[pl.ds(0, CE)], yb, sem).wait()

        def scatter_quads(eb, yb, nquads):
            def quad(q, mv):
                j = q * (4 * L)
                m = mv
                for u in range(4):
                    s = eb[0, pl.ds(j + u * L, L)]
                    d = eb[1, pl.ds(j + u * L, L)]
                    yv = yb[pl.ds(j + u * L, L)]
                    plsc.addupdate_scatter(acc, [d], yv)
                    plsc.addupdate_scatter(acc, [s], -yv)
                    m = jnp.maximum(m, jnp.maximum(s, d))
                return m

            maxb[...] = lax.fori_loop(0, nquads, quad, maxb[...])

        assert CE % (4 * L) == 0 and TAIL_PER_TILE % (4 * L) == 0
        start(0, ebuf0, ybuf0, sem0)
        start(1, ebuf1, ybuf1, sem1)

        @pl.loop(0, NCH, step=2)
        def _(c):
            wait(ebuf0, ybuf0, sem0)
            scatter_quads(ebuf0, ybuf0, CE // (4 * L))

            @pl.when(c + 2 < NCH)
            def _():
                start(c + 2, ebuf0, ybuf0, sem0)

            wait(ebuf1, ybuf1, sem1)
            scatter_quads(ebuf1, ybuf1, CE // (4 * L))

            @pl.when(c + 3 < NCH)
            def _():
                start(c + 3, ebuf1, ybuf1, sem1)

        @pl.when(wid < NS)
        def _():
            tbase = pl.multiple_of(TAIL_BASE + wid * TAIL_PER_TILE, 128)
            pltpu.sync_copy(ei_hbm.at[:, pl.ds(tbase, TAIL_PER_TILE)],
                            ebuf0.at[:, pl.ds(0, TAIL_PER_TILE)])
            pltpu.sync_copy(y_hbm.at[pl.ds(tbase, TAIL_PER_TILE)],
                            ybuf0.at[pl.ds(0, TAIL_PER_TILE)])
            scatter_quads(ebuf0, ybuf0, TAIL_PER_TILE // (4 * L))

        lo = sid * SLICE

        @pl.loop(0, SLICE, step=L)
        def _(i):
            ssum[pl.ds(i, L)] = acc[pl.ds(lo + i, L)]

        @pl.loop(1, NS)
        def _(r):
            dst_owner = lax.rem(sid + r, NS)
            pltpu.sync_copy(acc.at[pl.ds(dst_owner * SLICE, SLICE)],
                            shared.at[sid])
            plsc.subcore_barrier()
            src_slot = lax.rem(sid - r + NS, NS)
            pltpu.sync_copy(shared.at[src_slot], tmp)

            @pl.loop(0, SLICE, step=4 * L)
            def _(i):
                for u in range(4):
                    ssum[pl.ds(i + u * L, L)] += tmp[pl.ds(i + u * L, L)]

            plsc.subcore_barrier()

        pltpu.sync_copy(ssum, part_hbm.at[cid, pl.ds(lo, SLICE)])
        pltpu.sync_copy(maxb, max_hbm.at[wid])

    return k(edge_index, y)


def _tc_finalize(partials, maxes):
    def body(p_ref, m_ref, o_ref):
        a = p_ref[...]
        diff = a[: N_PAD // 128] + a[N_PAD // 128:]
        m = jnp.max(m_ref[...])
        o_ref[0, 0] = jnp.sum(jnp.abs(diff)) / (m.astype(jnp.float32) + 1.0)

    p2 = partials.reshape(NC * (N_PAD // 128), 128)
    m2 = maxes.reshape(NW * L // 128, 128)
    return pl.pallas_call(
        body,
        out_shape=jax.ShapeDtypeStruct((1, 1), jnp.float32),
        out_specs=pl.BlockSpec(memory_space=pltpu.SMEM),
    )(p2, m2)


def kernel(edge_index, y_hat):
    y = y_hat.reshape(-1)
    partials, maxes = _sc_scatter(edge_index, y)
    return _tc_finalize(partials, maxes)[0, 0]


# speedup vs baseline: 102.6174x; 1.1436x over previous
import dataclasses
import functools

import jax
import jax.numpy as jnp
from jax import lax
from jax.experimental import pallas as pl
from jax.experimental.pallas import tpu as pltpu
from jax.experimental.pallas import tpu_sc as plsc

N_PAD = 100352
NC, NS, L = 2, 16, 16
NW = NC * NS
E_TOTAL = 6400000
EPW = 199936
CE = 1408
NCH = EPW // CE
TAIL_BASE = NW * EPW
TAIL_PER_TILE = (E_TOTAL - TAIL_BASE) // L
SLICE = N_PAD // NS


def _sc_compiler_params():
    cp = pltpu.CompilerParams()
    if "needs_layout_passes" in pltpu.CompilerParams.__dataclass_fields__:
        cp = dataclasses.replace(cp, needs_layout_passes=False)
    return cp


def _sc_scatter(edge_index, y):
    mesh = plsc.VectorSubcoreMesh(core_axis_name="c", subcore_axis_name="s")

    @functools.partial(
        pl.kernel,
        compiler_params=_sc_compiler_params(),
        out_type=(
            jax.ShapeDtypeStruct((NC, N_PAD), jnp.float32),
            jax.ShapeDtypeStruct((NW, L), jnp.int32),
        ),
        mesh=mesh,
        scratch_types=[
            pltpu.VMEM((N_PAD,), jnp.float32),
            pltpu.VMEM((2, CE), jnp.int32),
            pltpu.VMEM((CE,), jnp.float32),
            pltpu.VMEM((2, CE), jnp.int32),
            pltpu.VMEM((CE,), jnp.float32),
            pltpu.VMEM((L,), jnp.int32),
            pltpu.VMEM((SLICE,), jnp.float32),
            pltpu.VMEM((SLICE,), jnp.float32),
            pltpu.VMEM_SHARED((NS, SLICE), jnp.float32),
            pltpu.SemaphoreType.DMA,
            pltpu.SemaphoreType.DMA,
        ],
    )
    def k(ei_hbm, y_hbm, part_hbm, max_hbm,
          acc, ebuf0, ybuf0, ebuf1, ybuf1,
          maxb, tmp, ssum, shared, sem0, sem1):
        cid = lax.axis_index("c")
        sid = lax.axis_index("s")
        wid = cid * NS + sid

        zero16 = jnp.zeros((L,), jnp.float32)

        @pl.loop(0, N_PAD, step=8 * L)
        def _(i):
            for u in range(8):
                acc[pl.ds(i + u * L, L)] = zero16

        maxb[...] = jnp.zeros((L,), jnp.int32)

        ebase = wid * EPW

        def start(c, eb, yb, sem):
            base = pl.multiple_of(ebase + c * CE, 128)
            pltpu.async_copy(ei_hbm.at[:, pl.ds(base, CE)], eb, sem)
            pltpu.async_copy(y_hbm.at[pl.ds(base, CE)], yb, sem)

        def wait(eb, yb, sem):
            pltpu.make_async_copy(ei_hbm.at[:, pl.ds(0, CE)], eb, sem).wait()
            pltpu.make_async_copy(y_hbm.at[pl.ds(0, CE)], yb, sem).wait()

        def scatter_quads(eb, yb, nquads):
            def group(j, mv):
                s = eb[0, pl.ds(j, L)]
                d = eb[1, pl.ds(j, L)]
                yv = yb[pl.ds(j, L)]
                plsc.addupdate_scatter(acc, [d], yv)
                plsc.addupdate_scatter(acc, [s], -yv)
                return jnp.maximum(mv, jnp.maximum(s, d))

            maxb[...] = plsc.parallel_loop(
                0, nquads * 4 * L, step=L, unroll=4, carry=maxb[...])(group)

        assert CE % (4 * L) == 0 and TAIL_PER_TILE % (4 * L) == 0
        start(0, ebuf0, ybuf0, sem0)
        start(1, ebuf1, ybuf1, sem1)

        @pl.loop(0, NCH, step=2)
        def _(c):
            wait(ebuf0, ybuf0, sem0)
            scatter_quads(ebuf0, ybuf0, CE // (4 * L))

            @pl.when(c + 2 < NCH)
            def _():
                start(c + 2, ebuf0, ybuf0, sem0)

            wait(ebuf1, ybuf1, sem1)
            scatter_quads(ebuf1, ybuf1, CE // (4 * L))

            @pl.when(c + 3 < NCH)
            def _():
                start(c + 3, ebuf1, ybuf1, sem1)

        @pl.when(wid < NS)
        def _():
            tbase = pl.multiple_of(TAIL_BASE + wid * TAIL_PER_TILE, 128)
            pltpu.sync_copy(ei_hbm.at[:, pl.ds(tbase, TAIL_PER_TILE)],
                            ebuf0.at[:, pl.ds(0, TAIL_PER_TILE)])
            pltpu.sync_copy(y_hbm.at[pl.ds(tbase, TAIL_PER_TILE)],
                            ybuf0.at[pl.ds(0, TAIL_PER_TILE)])
            scatter_quads(ebuf0, ybuf0, TAIL_PER_TILE // (4 * L))

        lo = sid * SLICE

        @pl.loop(0, SLICE, step=L)
        def _(i):
            ssum[pl.ds(i, L)] = acc[pl.ds(lo + i, L)]

        @pl.loop(1, NS)
        def _(r):
            dst_owner = lax.rem(sid + r, NS)
            pltpu.sync_copy(acc.at[pl.ds(dst_owner * SLICE, SLICE)],
                            shared.at[sid])
            plsc.subcore_barrier()
            src_slot = lax.rem(sid - r + NS, NS)
            pltpu.sync_copy(shared.at[src_slot], tmp)

            @pl.loop(0, SLICE, step=4 * L)
            def _(i):
                for u in range(4):
                    ssum[pl.ds(i + u * L, L)] += tmp[pl.ds(i + u * L, L)]

            plsc.subcore_barrier()

        pltpu.sync_copy(ssum, part_hbm.at[cid, pl.ds(lo, SLICE)])
        pltpu.sync_copy(maxb, max_hbm.at[wid])

    return k(edge_index, y)


def _tc_finalize(partials, maxes):
    def body(p_ref, m_ref, o_ref):
        a = p_ref[...]
        diff = a[: N_PAD // 128] + a[N_PAD // 128:]
        m = jnp.max(m_ref[...])
        o_ref[0, 0] = jnp.sum(jnp.abs(diff)) / (m.astype(jnp.float32) + 1.0)

    p2 = partials.reshape(NC * (N_PAD // 128), 128)
    m2 = maxes.reshape(NW * L // 128, 128)
    return pl.pallas_call(
        body,
        out_shape=jax.ShapeDtypeStruct((1, 1), jnp.float32),
        out_specs=pl.BlockSpec(memory_space=pltpu.SMEM),
    )(p2, m2)


def kernel(edge_index, y_hat):
    y = y_hat.reshape(-1)
    partials, maxes = _sc_scatter(edge_index, y)
    return _tc_finalize(partials, maxes)[0, 0]


# speedup vs baseline: 103.9932x; 1.0134x over previous
import dataclasses
import functools

import jax
import jax.numpy as jnp
from jax import lax
from jax.experimental import pallas as pl
from jax.experimental.pallas import tpu as pltpu
from jax.experimental.pallas import tpu_sc as plsc

N_PAD = 100352
NC, NS, L = 2, 16, 16
NW = NC * NS
E_TOTAL = 6400000
EPW = 199936
CE = 1408
NCH = EPW // CE
TAIL_BASE = NW * EPW
TAIL_PER_TILE = (E_TOTAL - TAIL_BASE) // L
SLICE = N_PAD // NS


def _sc_compiler_params():
    cp = pltpu.CompilerParams()
    if "needs_layout_passes" in pltpu.CompilerParams.__dataclass_fields__:
        cp = dataclasses.replace(cp, needs_layout_passes=False)
    return cp


def _sc_scatter(edge_index, y):
    mesh = plsc.VectorSubcoreMesh(core_axis_name="c", subcore_axis_name="s")

    @functools.partial(
        pl.kernel,
        compiler_params=_sc_compiler_params(),
        out_type=(
            jax.ShapeDtypeStruct((NW, N_PAD), jnp.float32),
            jax.ShapeDtypeStruct((NW, L), jnp.int32),
        ),
        mesh=mesh,
        scratch_types=[
            pltpu.VMEM((N_PAD,), jnp.float32),
            pltpu.VMEM((2, CE), jnp.int32),
            pltpu.VMEM((CE,), jnp.float32),
            pltpu.VMEM((2, CE), jnp.int32),
            pltpu.VMEM((CE,), jnp.float32),
            pltpu.VMEM((L,), jnp.int32),
            pltpu.SemaphoreType.DMA,
            pltpu.SemaphoreType.DMA,
        ],
    )
    def k(ei_hbm, y_hbm, part_hbm, max_hbm,
          acc, ebuf0, ybuf0, ebuf1, ybuf1,
          maxb, sem0, sem1):
        cid = lax.axis_index("c")
        sid = lax.axis_index("s")
        wid = cid * NS + sid

        zero16 = jnp.zeros((L,), jnp.float32)

        @pl.loop(0, N_PAD, step=8 * L)
        def _(i):
            for u in range(8):
                acc[pl.ds(i + u * L, L)] = zero16

        maxb[...] = jnp.zeros((L,), jnp.int32)

        ebase = wid * EPW

        def start(c, eb, yb, sem):
            base = pl.multiple_of(ebase + c * CE, 128)
            pltpu.async_copy(ei_hbm.at[:, pl.ds(base, CE)], eb, sem)
            pltpu.async_copy(y_hbm.at[pl.ds(base, CE)], yb, sem)

        def wait(eb, yb, sem):
            pltpu.make_async_copy(ei_hbm.at[:, pl.ds(0, CE)], eb, sem).wait()
            pltpu.make_async_copy(y_hbm.at[pl.ds(0, CE)], yb, sem).wait()

        def scatter_quads(eb, yb, nquads):
            def group(j, mv):
                s = eb[0, pl.ds(j, L)]
                d = eb[1, pl.ds(j, L)]
                yv = yb[pl.ds(j, L)]
                plsc.addupdate_scatter(acc, [d], yv)
                plsc.addupdate_scatter(acc, [s], -yv)
                return jnp.maximum(mv, jnp.maximum(s, d))

            maxb[...] = plsc.parallel_loop(
                0, nquads * 4 * L, step=L, unroll=4, carry=maxb[...])(group)

        assert CE % (4 * L) == 0 and TAIL_PER_TILE % (4 * L) == 0
        start(0, ebuf0, ybuf0, sem0)
        start(1, ebuf1, ybuf1, sem1)

        @pl.loop(0, NCH, step=2)
        def _(c):
            wait(ebuf0, ybuf0, sem0)
            scatter_quads(ebuf0, ybuf0, CE // (4 * L))

            @pl.when(c + 2 < NCH)
            def _():
                start(c + 2, ebuf0, ybuf0, sem0)

            wait(ebuf1, ybuf1, sem1)
            scatter_quads(ebuf1, ybuf1, CE // (4 * L))

            @pl.when(c + 3 < NCH)
            def _():
                start(c + 3, ebuf1, ybuf1, sem1)

        @pl.when(wid < NS)
        def _():
            tbase = pl.multiple_of(TAIL_BASE + wid * TAIL_PER_TILE, 128)
            pltpu.sync_copy(ei_hbm.at[:, pl.ds(tbase, TAIL_PER_TILE)],
                            ebuf0.at[:, pl.ds(0, TAIL_PER_TILE)])
            pltpu.sync_copy(y_hbm.at[pl.ds(tbase, TAIL_PER_TILE)],
                            ybuf0.at[pl.ds(0, TAIL_PER_TILE)])
            scatter_quads(ebuf0, ybuf0, TAIL_PER_TILE // (4 * L))

        pltpu.sync_copy(acc, part_hbm.at[wid])
        pltpu.sync_copy(maxb, max_hbm.at[wid])

    return k(edge_index, y)


def _tc_finalize(partials, maxes):
    rows = N_PAD // 128

    def body(p_ref, m_ref, o_ref):
        a = p_ref[...]
        diff = a[:rows]
        for w in range(1, NW):
            diff = diff + a[w * rows:(w + 1) * rows]
        m = jnp.max(m_ref[...])
        o_ref[0, 0] = jnp.sum(jnp.abs(diff)) / (m.astype(jnp.float32) + 1.0)

    p2 = partials.reshape(NW * rows, 128)
    m2 = maxes.reshape(NW * L // 128, 128)
    return pl.pallas_call(
        body,
        out_shape=jax.ShapeDtypeStruct((1, 1), jnp.float32),
        out_specs=pl.BlockSpec(memory_space=pltpu.SMEM),
    )(p2, m2)


def kernel(edge_index, y_hat):
    y = y_hat.reshape(-1)
    partials, maxes = _sc_scatter(edge_index, y)
    return _tc_finalize(partials, maxes)[0, 0]
